# Initial kernel scaffold; baseline (speedup 1.0000x reference)
import jax
import jax.numpy as jnp
from jax.experimental import pallas as pl


def kernel(Encoder, Neighbor_Index_Bond, Mask_Bond, Mask_NonBond, Neighbor_Type_Bond, Neighbor_Distance_NonBond, Solvent_Properties, Solvent_ID, params):
    raise NotImplementedError("write your pallas kernel here")


# speedup vs baseline: 15.8446x; 15.8446x over previous
import functools

import jax
import jax.numpy as jnp
import numpy as np
from jax.experimental import pallas as pl
from jax.experimental.pallas import tpu as pltpu

B, A, NBR = 32, 256, 48
NAB = 16
NG = 16
NF = 64
E = A * NBR
CUTOFF = 1.0
_LOG2 = float(np.log(2.0))


def _ssp(x):
    return jnp.maximum(x, 0.0) + jnp.log1p(jnp.exp(-jnp.abs(x))) - _LOG2


def _prep_kernel(emb, bond_w, bond_b, fw1, fb1, fw2, fb2,
                 sp, sid, lp, pampa, sw1, sb1, sw2, sb2, in2fs,
                 t2_out, y_out):
    t1 = (emb[...] @ bond_w[...] + bond_b[...]) @ fw1[...] + fb1[...]
    t2_out[...] = _ssp(t1) @ fw2[...] + fb2[...]
    s = sp[...]
    s = jnp.where(sid[...] == 103, lp[...], s)
    s = jnp.where(sid[...] == 104, pampa[...], s)
    z = _ssp(s @ sw1[...] + sb1[...])
    solv = z @ sw2[...] + sb2[...]
    y_out[...] = solv @ in2fs[...]


def _nonbond_kernel(ef, yrow, fw1s, fb1s, bd2, fb2s, bdf2, f2bs, dwv, dbs,
                    offs, v_out):
    d = ef[0, :, 0:1]
    m = ef[0, :, 1:2]
    width = CUTOFF / (NG - 1)
    coeff = -0.5 / (width * width)
    diff = d - offs[...]
    fij = jnp.exp(coeff * (diff * diff))
    h = jnp.dot(fij.astype(jnp.bfloat16), fw1s[...].astype(jnp.bfloat16),
                preferred_element_type=jnp.float32) + fb1s[...]
    hm = _ssp(h) * m
    s = hm.reshape(NBR, A, 3 * NF).sum(axis=0)
    msum = m.reshape(NBR, A, 1).sum(axis=0)
    c = s @ bd2[...] + msum * fb2s[...]
    z = c * yrow[0]
    u = _ssp(z @ bdf2[...] + f2bs[...])
    v_out[0] = u @ dwv[...] + dbs[...]


def _bond_kernel(enc, ef, vnb, t2, atom_w, atom_b, in2f, f2out_w, f2out_b,
                 dw, db, out_ref):
    atom = enc[0] @ atom_w[...] + atom_b[...]
    idxc = ef[0, :, 0:1]
    typec = ef[0, :, 1:2]
    maskc = ef[0, :, 2:3]
    iota18 = jax.lax.broadcasted_iota(jnp.int32, (E, 18), 1).astype(jnp.float32)
    oh18 = (typec == iota18).astype(jnp.bfloat16)
    wg = jnp.dot(oh18, t2[...].astype(jnp.bfloat16),
                 preferred_element_type=jnp.float32)
    wm = wg * maskc
    iota_a = jax.lax.broadcasted_iota(jnp.int32, (E, A), 1).astype(jnp.float32)
    oh = (idxc == iota_a).astype(jnp.bfloat16)
    for _ in range(3):
        y = atom @ in2f[...]
        yg = jnp.dot(oh, y.astype(jnp.bfloat16),
                     preferred_element_type=jnp.float32)
        agg = (yg * wm).reshape(NBR, A, NF).sum(axis=0)
        u = _ssp(agg @ f2out_w[...] + f2out_b[...])
        atom = atom + u @ dw[...] + db[...]
    out_ref[0] = atom * vnb[0]


def _row(x):
    return x.reshape(1, -1)


@jax.jit
def kernel(Encoder, Neighbor_Index_Bond, Mask_Bond, Mask_NonBond,
           Neighbor_Type_Bond, Neighbor_Distance_NonBond,
           Solvent_Properties, Solvent_ID, params):
    p = params
    bi = p['bond_int']
    f32 = jnp.float32

    def edge_major(x):
        return x.astype(f32).transpose(0, 2, 1).reshape(B, E, 1)

    ef_bond = jnp.concatenate(
        [edge_major(Neighbor_Index_Bond), edge_major(Neighbor_Type_Bond),
         edge_major(Mask_Bond)], axis=-1)
    ef_nb = jnp.concatenate(
        [edge_major(Neighbor_Distance_NonBond), edge_major(Mask_NonBond)],
        axis=-1)

    ints = [p['atom_int1'], p['atom_int2'], p['atom_int3']]
    fw1s = jnp.concatenate([q['fw1'] for q in ints], axis=1)
    fb1s = jnp.concatenate([_row(q['fb1']) for q in ints], axis=1)
    bd2 = jax.scipy.linalg.block_diag(*[q['fw2'] for q in ints])
    fb2s = jnp.concatenate([_row(q['fb2']) for q in ints], axis=1)
    bdf2 = jax.scipy.linalg.block_diag(*[q['f2out_w'] for q in ints])
    f2bs = jnp.concatenate([_row(q['f2out_b']) for q in ints], axis=1)
    dwv = jnp.concatenate([q['dw'] for q in ints], axis=0)
    dbs = _row(ints[0]['db'] + ints[1]['db'] + ints[2]['db'])
    in2fs = jnp.concatenate([q['in2f_w'] for q in ints], axis=1)
    offs = jnp.linspace(0.0, CUTOFF, NG).reshape(1, NG)

    t2, y_solv = pl.pallas_call(
        _prep_kernel,
        out_shape=(jax.ShapeDtypeStruct((18, NF), f32),
                   jax.ShapeDtypeStruct((B, 3 * NF), f32)),
    )(p['emb'], p['bond_w'], _row(p['bond_b']), bi['fw1'], _row(bi['fb1']),
      bi['fw2'], _row(bi['fb2']),
      Solvent_Properties, Solvent_ID.reshape(B, 1).astype(jnp.int32),
      _row(p['logP_prop']), _row(p['PAMPA']),
      p['solv_w1'], _row(p['solv_b1']), p['solv_w2'], _row(p['solv_b2']),
      in2fs)

    yrow = y_solv.reshape(B, 1, 3 * NF)

    bspec = lambda shape: pl.BlockSpec(shape, lambda b: (b, 0, 0))
    wspec = lambda w: pl.BlockSpec(w.shape, lambda b: (0,) * w.ndim)
    nb_weights = [fw1s, fb1s, bd2, fb2s, bdf2, f2bs, dwv, dbs, offs]
    v = pl.pallas_call(
        _nonbond_kernel,
        grid=(B,),
        in_specs=[bspec((1, E, 2)), bspec((1, 1, 3 * NF))]
                 + [wspec(w) for w in nb_weights],
        out_specs=bspec((1, A, NF)),
        out_shape=jax.ShapeDtypeStruct((B, A, NF), f32),
        compiler_params=pltpu.CompilerParams(
            dimension_semantics=("parallel",)),
    )(ef_nb, yrow, *nb_weights)

    b_weights = [t2, p['atom_w'], _row(p['atom_b']), bi['in2f_w'],
                 bi['f2out_w'], _row(bi['f2out_b']), bi['dw'], _row(bi['db'])]
    out = pl.pallas_call(
        _bond_kernel,
        grid=(B,),
        in_specs=[bspec((1, A, NAB)), bspec((1, E, 3)), bspec((1, A, NF))]
                 + [wspec(w) for w in b_weights],
        out_specs=bspec((1, A, NF)),
        out_shape=jax.ShapeDtypeStruct((B, A, NF), f32),
        compiler_params=pltpu.CompilerParams(
            dimension_semantics=("parallel",)),
    )(Encoder, ef_bond, v, *b_weights)
    return out


# speedup vs baseline: 52.0293x; 3.2837x over previous
import functools

import jax
import jax.numpy as jnp
import numpy as np
from jax.experimental import pallas as pl
from jax.experimental.pallas import tpu as pltpu

B, A, NBR = 32, 256, 48
NAB = 16
NG = 16
NF = 64
E = A * NBR
CUTOFF = 1.0
_LOG2 = float(np.log(2.0))
_BF = jnp.bfloat16
_W = CUTOFF / (NG - 1)
_COEFF = -0.5 / (_W * _W)


def _ssp(x):
    return jnp.log1p(jnp.exp(x)) - _LOG2


def _prep_kernel(emb, bond_w, bond_b, fw1, fb1, fw2, fb2,
                 sp, sid, lp, pampa, sw1, sb1, sw2, sb2, in2fs,
                 t2_out, y_out):
    t1 = (emb[...] @ bond_w[...] + bond_b[...]) @ fw1[...] + fb1[...]
    t2_out[...] = _ssp(t1) @ fw2[...] + fb2[...]
    s = sp[...]
    s = jnp.where(sid[...] == 103, lp[...], s)
    s = jnp.where(sid[...] == 104, pampa[...], s)
    z = _ssp(s @ sw1[...] + sb1[...])
    solv = z @ sw2[...] + sb2[...]
    y_out[...] = solv @ in2fs[...]


def _nonbond_kernel(drow, mrow, ycol, fw1at, offs_col, lane_col,
                    bd2t, fb2s_col, bdf2t, f2bs_col, dwvt, dbs_col, v_out):
    d = drow[0]
    m = mrow[0]
    diff = offs_col[...] - d
    g = jnp.exp(_COEFF * (diff * diff))
    fij = jnp.where(lane_col[...] == float(NG), 1.0, g)
    h = jnp.dot(fw1at[...], fij.astype(_BF),
                preferred_element_type=jnp.float32)
    t = 1.0 + m * jnp.exp(h)
    s = jnp.zeros((3 * NF, A), jnp.float32)
    for grp in range(4):
        pg = t[:, (12 * grp) * A:(12 * grp + 1) * A]
        for k in range(1, 12):
            base = (12 * grp + k) * A
            pg = pg * t[:, base:base + A]
        s = s + jnp.log(pg)
    msum = m[:, 0:A]
    for n in range(1, NBR):
        msum = msum + m[:, n * A:(n + 1) * A]
    s = s - _LOG2 * msum
    c = jnp.dot(bd2t[...], s,
                preferred_element_type=jnp.float32) + fb2s_col[...] * msum
    z = c * ycol[0]
    u = _ssp(jnp.dot(bdf2t[...], z,
                     preferred_element_type=jnp.float32) + f2bs_col[...])
    v_out[0] = jnp.dot(dwvt[...], u,
                       preferred_element_type=jnp.float32) + dbs_col[...]


def _bond_kernel(enct, idxrow, typerow, mrow, vnbt, iota_a_col, iota18_col,
                 t2t, atom_wt, ab_col, in2ft, f2outt, fob_col, dwt, db_col,
                 out_ref):
    atomt = jnp.dot(atom_wt[...], enct[0],
                    preferred_element_type=jnp.float32) + ab_col[...]
    one = jnp.ones((), _BF)
    zero = jnp.zeros((), _BF)
    oht = jnp.where(iota_a_col[...] == idxrow[0], one, zero)
    oht18 = jnp.where(iota18_col[...] == typerow[0], one, zero)
    wmt = jnp.dot(t2t[...], oht18,
                  preferred_element_type=jnp.float32) * mrow[0]
    for _ in range(3):
        yt = jnp.dot(in2ft[...], atomt,
                     preferred_element_type=jnp.float32).astype(_BF)
        ygt = jnp.dot(yt, oht,
                      preferred_element_type=jnp.float32)
        prod = ygt * wmt
        agg = prod[:, 0:A]
        for n in range(1, NBR):
            agg = agg + prod[:, n * A:(n + 1) * A]
        u = _ssp(jnp.dot(f2outt[...], agg,
                         preferred_element_type=jnp.float32) + fob_col[...])
        atomt = atomt + jnp.dot(dwt[...], u,
                                preferred_element_type=jnp.float32) + db_col[...]
    out_ref[0] = atomt * vnbt[0]


def _row3(x, dt):
    return x.astype(dt).transpose(0, 2, 1).reshape(B, 1, E)


def _col(x):
    return x.reshape(-1, 1)


@jax.jit
def kernel(Encoder, Neighbor_Index_Bond, Mask_Bond, Mask_NonBond,
           Neighbor_Type_Bond, Neighbor_Distance_NonBond,
           Solvent_Properties, Solvent_ID, params):
    p = params
    bi = p['bond_int']
    f32 = jnp.float32

    idxrow = _row3(Neighbor_Index_Bond, _BF)
    typerow = _row3(Neighbor_Type_Bond, _BF)
    mrow = _row3(Mask_Bond, f32)
    drow = _row3(Neighbor_Distance_NonBond, f32)
    mnbrow = _row3(Mask_NonBond, f32)
    enct = Encoder.transpose(0, 2, 1)

    iota_a_col = jnp.arange(A, dtype=_BF).reshape(A, 1)
    iota18_col = jnp.arange(18, dtype=_BF).reshape(18, 1)
    offs_col = jnp.concatenate(
        [jnp.linspace(0.0, CUTOFF, NG), jnp.zeros((1,), f32)]).reshape(NG + 1, 1)
    lane_col = jnp.arange(NG + 1, dtype=f32).reshape(NG + 1, 1)

    ints = [p['atom_int1'], p['atom_int2'], p['atom_int3']]
    fw1s = jnp.concatenate([q['fw1'] for q in ints], axis=1)
    fb1s = jnp.concatenate([q['fb1'] for q in ints]).reshape(1, 3 * NF)
    fw1at = jnp.concatenate([fw1s, fb1s], axis=0).T.astype(_BF)
    bd2t = jax.scipy.linalg.block_diag(*[q['fw2'].T for q in ints])
    fb2s_col = _col(jnp.concatenate([q['fb2'] for q in ints]))
    bdf2t = jax.scipy.linalg.block_diag(*[q['f2out_w'].T for q in ints])
    f2bs_col = _col(jnp.concatenate([q['f2out_b'] for q in ints]))
    dwvt = jnp.concatenate([q['dw'] for q in ints], axis=0).T
    dbs_col = _col(ints[0]['db'] + ints[1]['db'] + ints[2]['db'])
    in2fs = jnp.concatenate([q['in2f_w'] for q in ints], axis=1)

    t2, y_solv = pl.pallas_call(
        _prep_kernel,
        out_shape=(jax.ShapeDtypeStruct((18, NF), f32),
                   jax.ShapeDtypeStruct((B, 3 * NF), f32)),
    )(p['emb'], p['bond_w'], p['bond_b'].reshape(1, NF), bi['fw1'],
      bi['fb1'].reshape(1, NF), bi['fw2'], bi['fb2'].reshape(1, NF),
      Solvent_Properties, Solvent_ID.reshape(B, 1).astype(jnp.int32),
      p['logP_prop'].reshape(1, 5), p['PAMPA'].reshape(1, 5),
      p['solv_w1'], p['solv_b1'].reshape(1, NAB), p['solv_w2'],
      p['solv_b2'].reshape(1, NF), in2fs)

    t2t = t2.T.astype(_BF)
    ycol = y_solv.reshape(B, 3 * NF, 1)

    bspec = lambda shape: pl.BlockSpec(shape, lambda b: (b, 0, 0))
    wspec = lambda w: pl.BlockSpec(w.shape, lambda b: (0,) * w.ndim)
    nb_weights = [fw1at, offs_col, lane_col, bd2t, fb2s_col, bdf2t,
                  f2bs_col, dwvt, dbs_col]
    vt = pl.pallas_call(
        _nonbond_kernel,
        grid=(B,),
        in_specs=[bspec((1, 1, E)), bspec((1, 1, E)), bspec((1, 3 * NF, 1))]
                 + [wspec(w) for w in nb_weights],
        out_specs=bspec((1, NF, A)),
        out_shape=jax.ShapeDtypeStruct((B, NF, A), f32),
        compiler_params=pltpu.CompilerParams(
            dimension_semantics=("arbitrary",)),
    )(drow, mnbrow, ycol, *nb_weights)

    b_weights = [iota_a_col, iota18_col, t2t, p['atom_w'].T,
                 _col(p['atom_b']), bi['in2f_w'].T, bi['f2out_w'].T,
                 _col(bi['f2out_b']), bi['dw'].T, _col(bi['db'])]
    outt = pl.pallas_call(
        _bond_kernel,
        grid=(B,),
        in_specs=[bspec((1, NAB, A)), bspec((1, 1, E)), bspec((1, 1, E)),
                  bspec((1, 1, E)), bspec((1, NF, A))]
                 + [wspec(w) for w in b_weights],
        out_specs=bspec((1, NF, A)),
        out_shape=jax.ShapeDtypeStruct((B, NF, A), f32),
        compiler_params=pltpu.CompilerParams(
            dimension_semantics=("arbitrary",)),
    )(enct, idxrow, typerow, mrow, vt, *b_weights)
    return outt.transpose(0, 2, 1)
